# baseline (speedup 1.0000x reference)
import functools

import jax
import jax.numpy as jnp
from jax import lax
from jax.experimental import pallas as pl
from jax.experimental.pallas import tpu as pltpu
from jax.experimental.pallas import tpu_sc as plsc

N = 5000
E = 10000
IN_C = 128
HID = 64
EDGE_C = 16
OUT_C = 10
G = 250

NP = 5120
EP = 10240
NW = 32
BE = EP // NW
BN = NP // NW
EC = 8
FG = 4
ECW = EP // EC
SUB = 128
NEG = -3.0e38

def _wid():
    return lax.axis_index("s") * 2 + lax.axis_index("c")


@functools.cache
def _sc_kernels():
    mesh = plsc.VectorSubcoreMesh(core_axis_name="c", subcore_axis_name="s")

    def make_gather(C):
        @functools.partial(
            pl.kernel,
            mesh=mesh,
            out_type=jax.ShapeDtypeStruct((EP, C), jnp.float32),
            scratch_types=[
                pltpu.VMEM((2, BE // 2), jnp.int32),
                pltpu.VMEM((2, BE // 2, C), jnp.float32),
                pltpu.SemaphoreType.DMA,
                pltpu.SemaphoreType.DMA,
            ],
            compiler_params=pltpu.CompilerParams(use_tc_tiling_on_sc=False),
        )
        def gather(table_hbm, idx_hbm, out_hbm, idx_v, rows_v, sem, sem2):
            base = _wid() * BE
            hb = BE // 2
            pltpu.sync_copy(idx_hbm.at[pl.ds(base, hb)], idx_v.at[0])
            pltpu.sync_copy(idx_hbm.at[pl.ds(base + hb, hb)], idx_v.at[1])
            g0 = pltpu.async_copy(table_hbm.at[idx_v.at[0]], rows_v.at[0],
                                  sem)
            g1 = pltpu.async_copy(table_hbm.at[idx_v.at[1]], rows_v.at[1],
                                  sem)
            g0.wait()
            o0 = pltpu.async_copy(rows_v.at[0], out_hbm.at[pl.ds(base, hb)],
                                  sem2)
            g1.wait()
            o1 = pltpu.async_copy(rows_v.at[1],
                                  out_hbm.at[pl.ds(base + hb, hb)], sem2)
            o0.wait()
            o1.wait()

        return gather

    @functools.partial(
        pl.kernel,
        mesh=mesh,
        out_type=jax.ShapeDtypeStruct((EC, FG, NP * 16), jnp.float32),
        scratch_types=[
            pltpu.VMEM((ECW,), jnp.int32),
            pltpu.VMEM((NP * 16,), jnp.float32),
            pltpu.VMEM((2, SUB, HID), jnp.float32),
            pltpu.SemaphoreType.DMA,
        ],
    )
    def scatter_p1(msg_hbm, dst_hbm, part_hbm, dst_v, acc, msg_v, sem):
        w = _wid()
        ec = w // FG
        fg = w % FG
        nsub = ECW // SUB
        c_dst = pltpu.async_copy(dst_hbm.at[pl.ds(ec * ECW, ECW)], dst_v, sem)
        copies = [pltpu.async_copy(msg_hbm.at[pl.ds(ec * ECW, SUB)],
                                   msg_v.at[0], sem)]
        negv = jnp.full((16,), NEG, jnp.float32)

        def initb(i, carry):
            acc[pl.ds(i * 16, 16)] = negv
            return carry

        lax.fori_loop(0, NP, initb, 0, unroll=8)
        c_dst.wait()
        for s in range(nsub):
            if s + 1 < nsub:
                copies.append(pltpu.async_copy(
                    msg_hbm.at[pl.ds(ec * ECW + (s + 1) * SUB, SUB)],
                    msg_v.at[(s + 1) % 2], sem))
            copies[s].wait()

            def body(jg, carry, s=s):
                dvec = dst_v[pl.ds(s * SUB + jg * 16, 16)]
                for l in range(16):
                    d = dvec[l]
                    v = msg_v[s % 2, jg * 16 + l, pl.ds(fg * 16, 16)]
                    a = acc[pl.ds(d * 16, 16)]
                    acc[pl.ds(d * 16, 16)] = jnp.maximum(a, v)
                return carry

            lax.fori_loop(0, SUB // 16, body, 0)
        pltpu.sync_copy(acc, part_hbm.at[ec, fg])

    @functools.partial(
        pl.kernel,
        mesh=mesh,
        out_type=jax.ShapeDtypeStruct((NP, HID), jnp.float32),
        scratch_types=[
            pltpu.VMEM((FG * EC * BN * 16,), jnp.float32),
            pltpu.VMEM((BN, HID), jnp.float32),
            pltpu.VMEM((BN, HID), jnp.float32),
            pltpu.SemaphoreType.DMA,
        ],
    )
    def scatter_p2(part_hbm, rt_hbm, h_hbm, pbuf, rt_v, out_v, sem):
        r0 = _wid() * BN
        copies = [pltpu.async_copy(rt_hbm.at[pl.ds(r0, BN)], rt_v, sem)]
        for fg in range(FG):
            for ec in range(EC):
                copies.append(pltpu.async_copy(
                    part_hbm.at[ec, fg, pl.ds(r0 * 16, BN * 16)],
                    pbuf.at[pl.ds((fg * EC + ec) * BN * 16, BN * 16)], sem))
        for c in copies:
            c.wait()
        for fg in range(FG):

            def body(r, carry, fg=fg):
                base = fg * EC * BN * 16
                m = pbuf[pl.ds(base + r * 16, 16)]
                for ec in range(1, EC):
                    m = jnp.maximum(
                        m, pbuf[pl.ds(base + ec * BN * 16 + r * 16, 16)])
                m = jnp.where(m == NEG, 0.0, m)
                rt = rt_v[r, pl.ds(fg * 16, 16)]
                out_v[r, pl.ds(fg * 16, 16)] = jnp.maximum(m + rt, 0.0)
                return carry

            lax.fori_loop(0, BN, body, 0, unroll=4)
        pltpu.sync_copy(out_v, h_hbm.at[pl.ds(r0, BN)])

    return make_gather(IN_C), make_gather(HID), scatter_p1, scatter_p2


def _layer_body(xs_ref, ea_ref, hn_ref, w1_ref, b1_ref, w2p_ref, bb2_ref,
                root_ref, bias_ref, rep_ref, fold_ref, msg_ref, rt_ref):
    f32 = jnp.float32
    bf16 = jnp.bfloat16
    dot = lambda a, b: jnp.dot(a, b, preferred_element_type=f32)
    h = jnp.maximum(dot(ea_ref[...], w1_ref[...]) + b1_ref[...], 0.0)
    xs16 = xs_ref[...].astype(bf16)
    T = dot(xs16, w2p_ref[...]).astype(bf16)
    Z = dot(h.astype(bf16), rep_ref[...]).astype(bf16) * T
    w = HID * HID // 4
    Z2 = Z[:, :2 * w] + Z[:, 2 * w:]
    acc = Z2[:, :w].astype(f32) + Z2[:, w:].astype(f32)
    while w > HID:
        w //= 2
        acc = acc[:, :w] + acc[:, w:2 * w]
    msg_ref[...] = acc + dot(xs_ref[...], bb2_ref[...])
    rt_ref[...] = dot(hn_ref[...], root_ref[...]) + bias_ref[...]


TC_GRID = 16
TBE = EP // TC_GRID
TBN = NP // TC_GRID


def _make_layer(in_c):
    full = lambda shape: pl.BlockSpec(shape, lambda i: (0, 0))
    return pl.pallas_call(
        _layer_body,
        grid=(TC_GRID,),
        in_specs=[
            pl.BlockSpec((TBE, in_c), lambda i: (i, 0)),
            pl.BlockSpec((TBE, EDGE_C), lambda i: (i, 0)),
            pl.BlockSpec((TBN, in_c), lambda i: (i, 0)),
            full((EDGE_C, HID)),
            full((1, HID)),
            full((in_c, HID * HID)),
            full((in_c, HID)),
            full((in_c, HID)),
            full((1, HID)),
            full((HID, HID * HID)),
            full((HID * HID, HID)),
        ],
        out_specs=[
            pl.BlockSpec((TBE, HID), lambda i: (i, 0)),
            pl.BlockSpec((TBN, HID), lambda i: (i, 0)),
        ],
        out_shape=[
            jax.ShapeDtypeStruct((EP, HID), jnp.float32),
            jax.ShapeDtypeStruct((NP, HID), jnp.float32),
        ],
    )


_layer128 = _make_layer(IN_C)
_layer64 = _make_layer(HID)


def _head_body(h_ref, b_ref, w1_ref, b1_ref, w2_ref, b2_ref, out_ref):
    f32 = jnp.float32
    gids = lax.broadcasted_iota(jnp.int32, (G, NP), 0)
    oh = (gids == b_ref[...]).astype(f32)
    sums = jnp.dot(oh, h_ref[...], preferred_element_type=f32)
    cnt = jnp.sum(oh, axis=1, keepdims=True)
    pooled = sums / jnp.maximum(cnt, 1.0)
    z = jnp.maximum(
        jnp.dot(pooled, w1_ref[...], preferred_element_type=f32) + b1_ref[...],
        0.0)
    z = jnp.dot(z, w2_ref[...], preferred_element_type=f32) + b2_ref[...]
    z = z - jnp.max(z, axis=1, keepdims=True)
    out_ref[...] = z - jnp.log(jnp.sum(jnp.exp(z), axis=1, keepdims=True))


_head = pl.pallas_call(
    _head_body,
    grid=(1,),
    in_specs=[
        pl.BlockSpec((NP, HID), lambda i: (0, 0)),
        pl.BlockSpec((1, NP), lambda i: (0, 0)),
        pl.BlockSpec((HID, HID), lambda i: (0, 0)),
        pl.BlockSpec((1, HID), lambda i: (0, 0)),
        pl.BlockSpec((HID, OUT_C), lambda i: (0, 0)),
        pl.BlockSpec((1, OUT_C), lambda i: (0, 0)),
    ],
    out_specs=pl.BlockSpec((G, OUT_C), lambda i: (0, 0)),
    out_shape=jax.ShapeDtypeStruct((G, OUT_C), jnp.float32),
)


def _prep_w2(w2, b2, in_c):
    w2p = w2.reshape(HID, in_c, HID).transpose(1, 0, 2).reshape(in_c, HID * HID)
    return w2p.astype(jnp.bfloat16), b2.reshape(in_c, HID)


def _scatter(msg, dst_p, rt):
    _, _, scatter_p1, scatter_p2 = _sc_kernels()
    part = scatter_p1(msg, dst_p)
    return scatter_p2(part, rt)


def kernel(x, edge_index, edge_attr, batch, en1_w1, en1_b1, en1_w2, en1_b2,
           root1, bias1, en2_w1, en2_b1, en2_w2, en2_b2, root2, bias2, en3_w1,
           en3_b1, en3_w2, en3_b2, root3, bias3, fc1_w, fc1_b, fc2_w, fc2_b):
    src_p = jnp.pad(edge_index[0].astype(jnp.int32), (0, EP - E))
    dst_p = jnp.pad(edge_index[1].astype(jnp.int32), (0, EP - E),
                    constant_values=NP - 1)
    ea_p = jnp.pad(edge_attr, ((0, EP - E), (0, 0)))
    x_p = jnp.pad(x, ((0, NP - N), (0, 0)))
    b_p = jnp.pad(batch.astype(jnp.int32), (0, NP - N),
                  constant_values=G + 5).reshape(1, NP)

    w2p1, bb21 = _prep_w2(en1_w2, en1_b2, IN_C)
    w2p2, bb22 = _prep_w2(en2_w2, en2_b2, HID)
    w2p3, bb23 = _prep_w2(en3_w2, en3_b2, HID)
    eye = jnp.eye(HID, dtype=jnp.float32)
    rep = jnp.kron(eye, jnp.ones((1, HID), jnp.float32)).astype(jnp.bfloat16)
    fold = jnp.tile(eye, (HID, 1)).astype(jnp.bfloat16)

    _gather128, _gather64, _, _ = _sc_kernels()
    xs1 = _gather128(x_p, src_p)
    msg1, rt1 = _layer128(xs1, ea_p, x_p, en1_w1, en1_b1.reshape(1, HID),
                          w2p1, bb21, root1, bias1.reshape(1, HID), rep, fold)
    h1 = _scatter(msg1, dst_p, rt1)

    xs2 = _gather64(h1, src_p)
    msg2, rt2 = _layer64(xs2, ea_p, h1, en2_w1, en2_b1.reshape(1, HID),
                         w2p2, bb22, root2, bias2.reshape(1, HID), rep, fold)
    h2 = _scatter(msg2, dst_p, rt2)

    xs3 = _gather64(h2, src_p)
    msg3, rt3 = _layer64(xs3, ea_p, h2, en3_w1, en3_b1.reshape(1, HID),
                         w2p3, bb23, root3, bias3.reshape(1, HID), rep, fold)
    h3 = _scatter(msg3, dst_p, rt3)

    return _head(h3, b_p, fc1_w, fc1_b.reshape(1, HID), fc2_w,
                 fc2_b.reshape(1, OUT_C))


# speedup vs baseline: 1.0077x; 1.0077x over previous
import functools

import jax
import jax.numpy as jnp
from jax import lax
from jax.experimental import pallas as pl
from jax.experimental.pallas import tpu as pltpu
from jax.experimental.pallas import tpu_sc as plsc

N = 5000
E = 10000
IN_C = 128
HID = 64
EDGE_C = 16
OUT_C = 10
G = 250

NP = 5120
EP = 10240
NW = 32
BE = EP // NW
BN = NP // NW
EC = 8
FG = 4
ECW = EP // EC
SUB = 128
NEG = -3.0e38

def _wid():
    return lax.axis_index("s") * 2 + lax.axis_index("c")


@functools.cache
def _sc_kernels():
    mesh = plsc.VectorSubcoreMesh(core_axis_name="c", subcore_axis_name="s")

    def make_gather(C):
        @functools.partial(
            pl.kernel,
            mesh=mesh,
            out_type=jax.ShapeDtypeStruct((EP, C), jnp.float32),
            scratch_types=[
                pltpu.VMEM((2, BE // 2), jnp.int32),
                pltpu.VMEM((2, BE // 2, C), jnp.float32),
                pltpu.SemaphoreType.DMA,
                pltpu.SemaphoreType.DMA,
            ],
            compiler_params=pltpu.CompilerParams(use_tc_tiling_on_sc=False),
        )
        def gather(table_hbm, idx_hbm, out_hbm, idx_v, rows_v, sem, sem2):
            base = _wid() * BE
            hb = BE // 2
            pltpu.sync_copy(idx_hbm.at[pl.ds(base, hb)], idx_v.at[0])
            pltpu.sync_copy(idx_hbm.at[pl.ds(base + hb, hb)], idx_v.at[1])
            g0 = pltpu.async_copy(table_hbm.at[idx_v.at[0]], rows_v.at[0],
                                  sem)
            g1 = pltpu.async_copy(table_hbm.at[idx_v.at[1]], rows_v.at[1],
                                  sem)
            g0.wait()
            o0 = pltpu.async_copy(rows_v.at[0], out_hbm.at[pl.ds(base, hb)],
                                  sem2)
            g1.wait()
            o1 = pltpu.async_copy(rows_v.at[1],
                                  out_hbm.at[pl.ds(base + hb, hb)], sem2)
            o0.wait()
            o1.wait()

        return gather

    @functools.partial(
        pl.kernel,
        mesh=mesh,
        out_type=jax.ShapeDtypeStruct((EC, FG, NP * 16), jnp.float32),
        scratch_types=[
            pltpu.VMEM((ECW,), jnp.int32),
            pltpu.VMEM((NP * 16,), jnp.float32),
            pltpu.VMEM((2, SUB, HID), jnp.float32),
            pltpu.SemaphoreType.DMA,
        ],
    )
    def scatter_p1(msg_hbm, dst_hbm, part_hbm, dst_v, acc, msg_v, sem):
        w = _wid()
        ec = w // FG
        fg = w % FG
        nsub = ECW // SUB
        c_dst = pltpu.async_copy(dst_hbm.at[pl.ds(ec * ECW, ECW)], dst_v, sem)
        copies = [pltpu.async_copy(msg_hbm.at[pl.ds(ec * ECW, SUB)],
                                   msg_v.at[0], sem)]
        negv = jnp.full((16,), NEG, jnp.float32)

        def initb(i, carry):
            acc[pl.ds(i * 16, 16)] = negv
            return carry

        lax.fori_loop(0, NP, initb, 0, unroll=8)
        c_dst.wait()
        for s in range(nsub):
            if s + 1 < nsub:
                copies.append(pltpu.async_copy(
                    msg_hbm.at[pl.ds(ec * ECW + (s + 1) * SUB, SUB)],
                    msg_v.at[(s + 1) % 2], sem))
            copies[s].wait()

            def body(jg, carry, s=s):
                dvec = dst_v[pl.ds(s * SUB + jg * 16, 16)]
                for l in range(16):
                    d = dvec[l]
                    v = msg_v[s % 2, jg * 16 + l, pl.ds(fg * 16, 16)]
                    a = acc[pl.ds(d * 16, 16)]
                    acc[pl.ds(d * 16, 16)] = jnp.maximum(a, v)
                return carry

            lax.fori_loop(0, SUB // 16, body, 0)
        pltpu.sync_copy(acc, part_hbm.at[ec, fg])

    @functools.partial(
        pl.kernel,
        mesh=mesh,
        out_type=jax.ShapeDtypeStruct((NP, HID), jnp.float32),
        scratch_types=[
            pltpu.VMEM((FG * EC * BN * 16,), jnp.float32),
            pltpu.VMEM((BN, HID), jnp.float32),
            pltpu.VMEM((BN, HID), jnp.float32),
            pltpu.SemaphoreType.DMA,
        ],
    )
    def scatter_p2(part_hbm, rt_hbm, h_hbm, pbuf, rt_v, out_v, sem):
        r0 = _wid() * BN
        copies = [pltpu.async_copy(rt_hbm.at[pl.ds(r0, BN)], rt_v, sem)]
        for fg in range(FG):
            for ec in range(EC):
                copies.append(pltpu.async_copy(
                    part_hbm.at[ec, fg, pl.ds(r0 * 16, BN * 16)],
                    pbuf.at[pl.ds((fg * EC + ec) * BN * 16, BN * 16)], sem))
        for c in copies:
            c.wait()
        for fg in range(FG):

            def body(r, carry, fg=fg):
                base = fg * EC * BN * 16
                m = pbuf[pl.ds(base + r * 16, 16)]
                for ec in range(1, EC):
                    m = jnp.maximum(
                        m, pbuf[pl.ds(base + ec * BN * 16 + r * 16, 16)])
                m = jnp.where(m == NEG, 0.0, m)
                rt = rt_v[r, pl.ds(fg * 16, 16)]
                out_v[r, pl.ds(fg * 16, 16)] = jnp.maximum(m + rt, 0.0)
                return carry

            lax.fori_loop(0, BN, body, 0)
        pltpu.sync_copy(out_v, h_hbm.at[pl.ds(r0, BN)])

    return make_gather(IN_C), make_gather(HID), scatter_p1, scatter_p2


def _layer_body(xs_ref, ea_ref, hn_ref, w1_ref, b1_ref, w2p_ref, bb2_ref,
                root_ref, bias_ref, rep_ref, fold_ref, msg_ref, rt_ref):
    f32 = jnp.float32
    bf16 = jnp.bfloat16
    dot = lambda a, b: jnp.dot(a, b, preferred_element_type=f32)
    h = jnp.maximum(dot(ea_ref[...], w1_ref[...]) + b1_ref[...], 0.0)
    xs16 = xs_ref[...].astype(bf16)
    T = dot(xs16, w2p_ref[...]).astype(bf16)
    Z = dot(h.astype(bf16), rep_ref[...]).astype(bf16) * T
    w = HID * HID // 4
    Z2 = Z[:, :2 * w] + Z[:, 2 * w:]
    acc = Z2[:, :w].astype(f32) + Z2[:, w:].astype(f32)
    while w > HID:
        w //= 2
        acc = acc[:, :w] + acc[:, w:2 * w]
    msg_ref[...] = acc + dot(xs_ref[...], bb2_ref[...])
    rt_ref[...] = dot(hn_ref[...], root_ref[...]) + bias_ref[...]


TC_GRID = 16
TBE = EP // TC_GRID
TBN = NP // TC_GRID


def _make_layer(in_c):
    full = lambda shape: pl.BlockSpec(shape, lambda i: (0, 0))
    return pl.pallas_call(
        _layer_body,
        grid=(TC_GRID,),
        in_specs=[
            pl.BlockSpec((TBE, in_c), lambda i: (i, 0)),
            pl.BlockSpec((TBE, EDGE_C), lambda i: (i, 0)),
            pl.BlockSpec((TBN, in_c), lambda i: (i, 0)),
            full((EDGE_C, HID)),
            full((1, HID)),
            full((in_c, HID * HID)),
            full((in_c, HID)),
            full((in_c, HID)),
            full((1, HID)),
            full((HID, HID * HID)),
            full((HID * HID, HID)),
        ],
        out_specs=[
            pl.BlockSpec((TBE, HID), lambda i: (i, 0)),
            pl.BlockSpec((TBN, HID), lambda i: (i, 0)),
        ],
        out_shape=[
            jax.ShapeDtypeStruct((EP, HID), jnp.float32),
            jax.ShapeDtypeStruct((NP, HID), jnp.float32),
        ],
    )


_layer128 = _make_layer(IN_C)
_layer64 = _make_layer(HID)


def _head_body(h_ref, b_ref, w1_ref, b1_ref, w2_ref, b2_ref, out_ref):
    f32 = jnp.float32
    gids = lax.broadcasted_iota(jnp.int32, (G, NP), 0)
    oh = (gids == b_ref[...]).astype(f32)
    sums = jnp.dot(oh, h_ref[...], preferred_element_type=f32)
    cnt = jnp.sum(oh, axis=1, keepdims=True)
    pooled = sums / jnp.maximum(cnt, 1.0)
    z = jnp.maximum(
        jnp.dot(pooled, w1_ref[...], preferred_element_type=f32) + b1_ref[...],
        0.0)
    z = jnp.dot(z, w2_ref[...], preferred_element_type=f32) + b2_ref[...]
    z = z - jnp.max(z, axis=1, keepdims=True)
    out_ref[...] = z - jnp.log(jnp.sum(jnp.exp(z), axis=1, keepdims=True))


_head = pl.pallas_call(
    _head_body,
    grid=(1,),
    in_specs=[
        pl.BlockSpec((NP, HID), lambda i: (0, 0)),
        pl.BlockSpec((1, NP), lambda i: (0, 0)),
        pl.BlockSpec((HID, HID), lambda i: (0, 0)),
        pl.BlockSpec((1, HID), lambda i: (0, 0)),
        pl.BlockSpec((HID, OUT_C), lambda i: (0, 0)),
        pl.BlockSpec((1, OUT_C), lambda i: (0, 0)),
    ],
    out_specs=pl.BlockSpec((G, OUT_C), lambda i: (0, 0)),
    out_shape=jax.ShapeDtypeStruct((G, OUT_C), jnp.float32),
)


def _prep_w2(w2, b2, in_c):
    w2p = w2.reshape(HID, in_c, HID).transpose(1, 0, 2).reshape(in_c, HID * HID)
    return w2p.astype(jnp.bfloat16), b2.reshape(in_c, HID)


def _scatter(msg, dst_p, rt):
    _, _, scatter_p1, scatter_p2 = _sc_kernels()
    part = scatter_p1(msg, dst_p)
    return scatter_p2(part, rt)


def kernel(x, edge_index, edge_attr, batch, en1_w1, en1_b1, en1_w2, en1_b2,
           root1, bias1, en2_w1, en2_b1, en2_w2, en2_b2, root2, bias2, en3_w1,
           en3_b1, en3_w2, en3_b2, root3, bias3, fc1_w, fc1_b, fc2_w, fc2_b):
    src_p = jnp.pad(edge_index[0].astype(jnp.int32), (0, EP - E))
    dst_p = jnp.pad(edge_index[1].astype(jnp.int32), (0, EP - E),
                    constant_values=NP - 1)
    ea_p = jnp.pad(edge_attr, ((0, EP - E), (0, 0)))
    x_p = jnp.pad(x, ((0, NP - N), (0, 0)))
    b_p = jnp.pad(batch.astype(jnp.int32), (0, NP - N),
                  constant_values=G + 5).reshape(1, NP)

    w2p1, bb21 = _prep_w2(en1_w2, en1_b2, IN_C)
    w2p2, bb22 = _prep_w2(en2_w2, en2_b2, HID)
    w2p3, bb23 = _prep_w2(en3_w2, en3_b2, HID)
    eye = jnp.eye(HID, dtype=jnp.float32)
    rep = jnp.kron(eye, jnp.ones((1, HID), jnp.float32)).astype(jnp.bfloat16)
    fold = jnp.tile(eye, (HID, 1)).astype(jnp.bfloat16)

    _gather128, _gather64, _, _ = _sc_kernels()
    xs1 = _gather128(x_p, src_p)
    msg1, rt1 = _layer128(xs1, ea_p, x_p, en1_w1, en1_b1.reshape(1, HID),
                          w2p1, bb21, root1, bias1.reshape(1, HID), rep, fold)
    h1 = _scatter(msg1, dst_p, rt1)

    xs2 = _gather64(h1, src_p)
    msg2, rt2 = _layer64(xs2, ea_p, h1, en2_w1, en2_b1.reshape(1, HID),
                         w2p2, bb22, root2, bias2.reshape(1, HID), rep, fold)
    h2 = _scatter(msg2, dst_p, rt2)

    xs3 = _gather64(h2, src_p)
    msg3, rt3 = _layer64(xs3, ea_p, h2, en3_w1, en3_b1.reshape(1, HID),
                         w2p3, bb23, root3, bias3.reshape(1, HID), rep, fold)
    h3 = _scatter(msg3, dst_p, rt3)

    return _head(h3, b_p, fc1_w, fc1_b.reshape(1, HID), fc2_w,
                 fc2_b.reshape(1, OUT_C))


# speedup vs baseline: 1.0168x; 1.0090x over previous
import functools

import jax
import jax.numpy as jnp
from jax import lax
from jax.experimental import pallas as pl
from jax.experimental.pallas import tpu as pltpu
from jax.experimental.pallas import tpu_sc as plsc

N = 5000
E = 10000
IN_C = 128
HID = 64
EDGE_C = 16
OUT_C = 10
G = 250

NP = 5120
EP = 10240
NW = 32
BE = EP // NW
BN = NP // NW
EC = 8
FG = 4
ECW = EP // EC
SUB = 128
NEG = -3.0e38

def _wid():
    return lax.axis_index("s") * 2 + lax.axis_index("c")


@functools.cache
def _sc_kernels():
    mesh = plsc.VectorSubcoreMesh(core_axis_name="c", subcore_axis_name="s")

    def make_gather(C):
        @functools.partial(
            pl.kernel,
            mesh=mesh,
            out_type=jax.ShapeDtypeStruct((EP, C), jnp.float32),
            scratch_types=[
                pltpu.VMEM((2, BE // 2), jnp.int32),
                pltpu.VMEM((2, BE // 2, C), jnp.float32),
                pltpu.SemaphoreType.DMA,
                pltpu.SemaphoreType.DMA,
            ],
            compiler_params=pltpu.CompilerParams(use_tc_tiling_on_sc=False),
        )
        def gather(table_hbm, idx_hbm, out_hbm, idx_v, rows_v, sem, sem2):
            base = _wid() * BE
            hb = BE // 2
            pltpu.sync_copy(idx_hbm.at[pl.ds(base, hb)], idx_v.at[0])
            pltpu.sync_copy(idx_hbm.at[pl.ds(base + hb, hb)], idx_v.at[1])
            g0 = pltpu.async_copy(table_hbm.at[idx_v.at[0]], rows_v.at[0],
                                  sem)
            g1 = pltpu.async_copy(table_hbm.at[idx_v.at[1]], rows_v.at[1],
                                  sem)
            g0.wait()
            o0 = pltpu.async_copy(rows_v.at[0], out_hbm.at[pl.ds(base, hb)],
                                  sem2)
            g1.wait()
            o1 = pltpu.async_copy(rows_v.at[1],
                                  out_hbm.at[pl.ds(base + hb, hb)], sem2)
            o0.wait()
            o1.wait()

        return gather

    @functools.partial(
        pl.kernel,
        mesh=mesh,
        out_type=jax.ShapeDtypeStruct((EC, FG, NP * 16), jnp.float32),
        scratch_types=[
            pltpu.VMEM((ECW,), jnp.int32),
            pltpu.VMEM((NP * 16,), jnp.float32),
            pltpu.VMEM((2, SUB, HID), jnp.float32),
            pltpu.SemaphoreType.DMA,
        ],
    )
    def scatter_p1(msg_hbm, dst_hbm, part_hbm, dst_v, acc, msg_v, sem):
        w = _wid()
        ec = w // FG
        fg = w % FG
        nsub = ECW // SUB
        c_dst = pltpu.async_copy(dst_hbm.at[pl.ds(ec * ECW, ECW)], dst_v, sem)
        copies = [pltpu.async_copy(msg_hbm.at[pl.ds(ec * ECW, SUB)],
                                   msg_v.at[0], sem)]
        negv = jnp.full((16,), NEG, jnp.float32)

        def initb(i, carry):
            acc[pl.ds(i * 16, 16)] = negv
            return carry

        lax.fori_loop(0, NP, initb, 0, unroll=8)
        c_dst.wait()
        for s in range(nsub):
            if s + 1 < nsub:
                copies.append(pltpu.async_copy(
                    msg_hbm.at[pl.ds(ec * ECW + (s + 1) * SUB, SUB)],
                    msg_v.at[(s + 1) % 2], sem))
            copies[s].wait()

            def body(jg, carry, s=s):
                dvec = dst_v[pl.ds(s * SUB + jg * 16, 16)]
                for l in range(16):
                    d = dvec[l]
                    v = msg_v[s % 2, jg * 16 + l, pl.ds(fg * 16, 16)]
                    a = acc[pl.ds(d * 16, 16)]
                    acc[pl.ds(d * 16, 16)] = jnp.maximum(a, v)
                return carry

            lax.fori_loop(0, SUB // 16, body, 0)
        pltpu.sync_copy(acc, part_hbm.at[ec, fg])

    @functools.partial(
        pl.kernel,
        mesh=mesh,
        out_type=jax.ShapeDtypeStruct((NP, HID), jnp.float32),
        scratch_types=[
            pltpu.VMEM((FG * EC * BN * 16,), jnp.float32),
            pltpu.VMEM((BN, HID), jnp.float32),
            pltpu.VMEM((BN, HID), jnp.float32),
            pltpu.SemaphoreType.DMA,
        ],
    )
    def scatter_p2(part_hbm, rt_hbm, h_hbm, pbuf, rt_v, out_v, sem):
        r0 = _wid() * BN
        copies = [pltpu.async_copy(rt_hbm.at[pl.ds(r0, BN)], rt_v, sem)]
        for fg in range(FG):
            for ec in range(EC):
                copies.append(pltpu.async_copy(
                    part_hbm.at[ec, fg, pl.ds(r0 * 16, BN * 16)],
                    pbuf.at[pl.ds((fg * EC + ec) * BN * 16, BN * 16)], sem))
        for c in copies:
            c.wait()
        for fg in range(FG):

            def body(r, carry, fg=fg):
                base = fg * EC * BN * 16
                m = pbuf[pl.ds(base + r * 16, 16)]
                for ec in range(1, EC):
                    m = jnp.maximum(
                        m, pbuf[pl.ds(base + ec * BN * 16 + r * 16, 16)])
                m = jnp.where(m == NEG, 0.0, m)
                rt = rt_v[r, pl.ds(fg * 16, 16)]
                out_v[r, pl.ds(fg * 16, 16)] = jnp.maximum(m + rt, 0.0)
                return carry

            lax.fori_loop(0, BN, body, 0)
        pltpu.sync_copy(out_v, h_hbm.at[pl.ds(r0, BN)])

    return make_gather(IN_C), make_gather(HID), scatter_p1, scatter_p2


def _layer_body(xs_ref, ea_ref, hn_ref, w1_ref, b1_ref, w2p_ref, bb2_ref,
                root_ref, bias_ref, rep_ref, msg_ref, rt_ref):
    f32 = jnp.float32
    bf16 = jnp.bfloat16
    dot = lambda a, b: jnp.dot(a, b, preferred_element_type=f32)
    h = jnp.maximum(dot(ea_ref[...], w1_ref[...]) + b1_ref[...], 0.0)
    xs16 = xs_ref[...].astype(bf16)
    T = dot(xs16, w2p_ref[...]).astype(bf16)
    Z = dot(h.astype(bf16), rep_ref[...]).astype(bf16) * T
    w = HID * HID // 4
    Z2 = Z[:, :2 * w] + Z[:, 2 * w:]
    acc = Z2[:, :w].astype(f32) + Z2[:, w:].astype(f32)
    while w > HID:
        w //= 2
        acc = acc[:, :w] + acc[:, w:2 * w]
    msg_ref[...] = acc + dot(xs_ref[...], bb2_ref[...])
    rt_ref[...] = dot(hn_ref[...], root_ref[...]) + bias_ref[...]


TC_GRID = 16
TBE = EP // TC_GRID
TBN = NP // TC_GRID


def _make_layer(in_c):
    full = lambda shape: pl.BlockSpec(shape, lambda i: (0, 0))
    return pl.pallas_call(
        _layer_body,
        grid=(TC_GRID,),
        in_specs=[
            pl.BlockSpec((TBE, in_c), lambda i: (i, 0)),
            pl.BlockSpec((TBE, EDGE_C), lambda i: (i, 0)),
            pl.BlockSpec((TBN, in_c), lambda i: (i, 0)),
            full((EDGE_C, HID)),
            full((1, HID)),
            full((in_c, HID * HID)),
            full((in_c, HID)),
            full((in_c, HID)),
            full((1, HID)),
            full((HID, HID * HID)),
        ],
        out_specs=[
            pl.BlockSpec((TBE, HID), lambda i: (i, 0)),
            pl.BlockSpec((TBN, HID), lambda i: (i, 0)),
        ],
        out_shape=[
            jax.ShapeDtypeStruct((EP, HID), jnp.float32),
            jax.ShapeDtypeStruct((NP, HID), jnp.float32),
        ],
    )


_layer128 = _make_layer(IN_C)
_layer64 = _make_layer(HID)


def _head_body(h_ref, b_ref, w1_ref, b1_ref, w2_ref, b2_ref, out_ref):
    f32 = jnp.float32
    gids = lax.broadcasted_iota(jnp.int32, (G, NP), 0)
    oh = (gids == b_ref[...]).astype(f32)
    sums = jnp.dot(oh, h_ref[...], preferred_element_type=f32)
    cnt = jnp.sum(oh, axis=1, keepdims=True)
    pooled = sums / jnp.maximum(cnt, 1.0)
    z = jnp.maximum(
        jnp.dot(pooled, w1_ref[...], preferred_element_type=f32) + b1_ref[...],
        0.0)
    z = jnp.dot(z, w2_ref[...], preferred_element_type=f32) + b2_ref[...]
    z = z - jnp.max(z, axis=1, keepdims=True)
    out_ref[...] = z - jnp.log(jnp.sum(jnp.exp(z), axis=1, keepdims=True))


_head = pl.pallas_call(
    _head_body,
    grid=(1,),
    in_specs=[
        pl.BlockSpec((NP, HID), lambda i: (0, 0)),
        pl.BlockSpec((1, NP), lambda i: (0, 0)),
        pl.BlockSpec((HID, HID), lambda i: (0, 0)),
        pl.BlockSpec((1, HID), lambda i: (0, 0)),
        pl.BlockSpec((HID, OUT_C), lambda i: (0, 0)),
        pl.BlockSpec((1, OUT_C), lambda i: (0, 0)),
    ],
    out_specs=pl.BlockSpec((G, OUT_C), lambda i: (0, 0)),
    out_shape=jax.ShapeDtypeStruct((G, OUT_C), jnp.float32),
)


def _prep_w2(w2, b2, in_c):
    w2p = w2.reshape(HID, in_c, HID).transpose(1, 0, 2).reshape(in_c, HID * HID)
    return w2p.astype(jnp.bfloat16), b2.reshape(in_c, HID)


def _scatter(msg, dst_p, rt):
    _, _, scatter_p1, scatter_p2 = _sc_kernels()
    part = scatter_p1(msg, dst_p)
    return scatter_p2(part, rt)


def kernel(x, edge_index, edge_attr, batch, en1_w1, en1_b1, en1_w2, en1_b2,
           root1, bias1, en2_w1, en2_b1, en2_w2, en2_b2, root2, bias2, en3_w1,
           en3_b1, en3_w2, en3_b2, root3, bias3, fc1_w, fc1_b, fc2_w, fc2_b):
    src_p = jnp.pad(edge_index[0].astype(jnp.int32), (0, EP - E))
    dst_p = jnp.pad(edge_index[1].astype(jnp.int32), (0, EP - E),
                    constant_values=NP - 1)
    ea_p = jnp.pad(edge_attr, ((0, EP - E), (0, 0)))
    x_p = jnp.pad(x, ((0, NP - N), (0, 0)))
    b_p = jnp.pad(batch.astype(jnp.int32), (0, NP - N),
                  constant_values=G + 5).reshape(1, NP)

    w2p1, bb21 = _prep_w2(en1_w2, en1_b2, IN_C)
    w2p2, bb22 = _prep_w2(en2_w2, en2_b2, HID)
    w2p3, bb23 = _prep_w2(en3_w2, en3_b2, HID)
    eye = jnp.eye(HID, dtype=jnp.float32)
    rep = jnp.kron(eye, jnp.ones((1, HID), jnp.float32)).astype(jnp.bfloat16)

    _gather128, _gather64, _, _ = _sc_kernels()
    xs1 = _gather128(x_p, src_p)
    msg1, rt1 = _layer128(xs1, ea_p, x_p, en1_w1, en1_b1.reshape(1, HID),
                          w2p1, bb21, root1, bias1.reshape(1, HID), rep)
    h1 = _scatter(msg1, dst_p, rt1)

    xs2 = _gather64(h1, src_p)
    msg2, rt2 = _layer64(xs2, ea_p, h1, en2_w1, en2_b1.reshape(1, HID),
                         w2p2, bb22, root2, bias2.reshape(1, HID), rep)
    h2 = _scatter(msg2, dst_p, rt2)

    xs3 = _gather64(h2, src_p)
    msg3, rt3 = _layer64(xs3, ea_p, h2, en3_w1, en3_b1.reshape(1, HID),
                         w2p3, bb23, root3, bias3.reshape(1, HID), rep)
    h3 = _scatter(msg3, dst_p, rt3)

    return _head(h3, b_p, fc1_w, fc1_b.reshape(1, HID), fc2_w,
                 fc2_b.reshape(1, OUT_C))
